# baseline (speedup 1.0000x reference)
import functools

import jax
import jax.numpy as jnp
from jax import lax
from jax.experimental import pallas as pl
from jax.experimental.pallas import tpu as pltpu
from jax.experimental.pallas import tpu_sc as plsc

_NC, _NS = 2, 16
_NW = _NC * _NS

_CAT_CH = 128
_TXT_CH = 96


def _sc_gather_build(n_cat, n_txt, D):
    cat_per_w = n_cat // _NW
    txt_per_w = n_txt // _NW
    cat_iters = cat_per_w // _CAT_CH
    txt_iters = txt_per_w // _TXT_CH
    mesh = plsc.VectorSubcoreMesh(core_axis_name="c", subcore_axis_name="s")

    @functools.partial(
        pl.kernel,
        mesh=mesh,
        compiler_params=pltpu.CompilerParams(use_tc_tiling_on_sc=False),
        out_type=(
            jax.ShapeDtypeStruct((n_cat, D), jnp.float32),
            jax.ShapeDtypeStruct((n_txt, D), jnp.float32),
        ),
        scratch_types=[
            pltpu.VMEM((_CAT_CH,), jnp.int32),
            pltpu.VMEM((_CAT_CH, D), jnp.float32),
            pltpu.VMEM((_TXT_CH,), jnp.int32),
            pltpu.VMEM((_TXT_CH, D), jnp.float32),
            pltpu.SemaphoreType.DMA,
        ],
    )
    def sc_kernel(cat_idx, txt_idx, wcat, wnlp, out_cat, out_txt,
                  ic, rc, it, rt, sem):
        wid = lax.axis_index("s") * _NC + lax.axis_index("c")
        cat_base = wid * cat_per_w
        txt_base = wid * txt_per_w

        def cat_body(i, carry):
            o = cat_base + i * _CAT_CH
            pltpu.sync_copy(cat_idx.at[pl.ds(o, _CAT_CH)], ic)
            pltpu.async_copy(wcat.at[ic], rc, sem).wait()
            pltpu.sync_copy(rc, out_cat.at[pl.ds(o, _CAT_CH)])
            return carry

        def txt_body(i, carry):
            o = txt_base + i * _TXT_CH
            pltpu.sync_copy(txt_idx.at[pl.ds(o, _TXT_CH)], it)
            pltpu.async_copy(wnlp.at[it], rt, sem).wait()
            pltpu.sync_copy(rt, out_txt.at[pl.ds(o, _TXT_CH)])
            return carry

        lax.fori_loop(0, cat_iters, cat_body, 0)
        lax.fori_loop(0, txt_iters, txt_body, 0)

    return sc_kernel


def _dense_body(t_ref, w_ref, b_ref, g_ref, og_ref, ot_ref):
    og_ref[...] = jnp.broadcast_to(g_ref[...], og_ref.shape)
    ot_ref[...] = t_ref[...] * w_ref[...] + b_ref[...]


def kernel(target, cat_feat, text, global_token, W_lin, b_lin, W_cat, W_nlp):
    B, S, _ = target.shape
    D = global_token.shape[-1]
    N = B * S
    NT = B * (S + 1)

    gt2 = global_token.reshape(1, D).astype(jnp.float32)
    w2 = W_lin.reshape(1, D).astype(jnp.float32)
    b2 = b_lin.reshape(1, D).astype(jnp.float32)

    cat_idx = cat_feat.reshape(N).astype(jnp.int32)
    wnlp_ext = jnp.concatenate([W_nlp.astype(jnp.float32), gt2], axis=0)
    gt_row = jnp.full((B, 1), W_nlp.shape[0], dtype=jnp.int32)
    txt_idx = jnp.concatenate(
        [gt_row, text.astype(jnp.int32)], axis=1).reshape(NT)

    out_cat_flat, out_txt_flat = _sc_gather_build(N, NT, D)(
        cat_idx, txt_idx, W_cat, wnlp_ext)

    BLK = 2048
    og_flat, ot_flat = pl.pallas_call(
        _dense_body,
        grid=(N // BLK,),
        in_specs=[
            pl.BlockSpec((BLK, 1), lambda i: (i, 0)),
            pl.BlockSpec((1, D), lambda i: (0, 0)),
            pl.BlockSpec((1, D), lambda i: (0, 0)),
            pl.BlockSpec((1, D), lambda i: (0, 0)),
        ],
        out_specs=[
            pl.BlockSpec((BLK, D), lambda i: (i, 0)),
            pl.BlockSpec((BLK, D), lambda i: (i, 0)),
        ],
        out_shape=[
            jax.ShapeDtypeStruct((N, D), jnp.float32),
            jax.ShapeDtypeStruct((N, D), jnp.float32),
        ],
    )(target.reshape(N, 1), w2, b2, gt2)

    out_global = og_flat.reshape(B, S, D)
    out_target = ot_flat.reshape(B, S, D)
    out_cat = out_cat_flat.reshape(B, S, D)
    out_text = out_txt_flat.reshape(B, S + 1, D)
    return (out_global, out_target, out_cat, out_text)


# speedup vs baseline: 1.0179x; 1.0179x over previous
import functools

import jax
import jax.numpy as jnp
from jax import lax
from jax.experimental import pallas as pl
from jax.experimental.pallas import tpu as pltpu
from jax.experimental.pallas import tpu_sc as plsc

_NC, _NS = 2, 16
_NW = _NC * _NS

_CAT_CH = 128
_TXT_CH = 96


_NBUF = 3


def _sc_gather_build(n_cat, n_txt, D):
    cat_per_w = n_cat // _NW
    txt_per_w = n_txt // _NW
    cat_iters = cat_per_w // _CAT_CH
    txt_iters = txt_per_w // _TXT_CH
    mesh = plsc.VectorSubcoreMesh(core_axis_name="c", subcore_axis_name="s")

    @functools.partial(
        pl.kernel,
        mesh=mesh,
        compiler_params=pltpu.CompilerParams(use_tc_tiling_on_sc=False),
        out_type=(
            jax.ShapeDtypeStruct((n_cat, D), jnp.float32),
            jax.ShapeDtypeStruct((n_txt, D), jnp.float32),
        ),
        scratch_types=[
            pltpu.VMEM((cat_per_w,), jnp.int32),
            pltpu.VMEM((txt_per_w,), jnp.int32),
        ]
        + [pltpu.VMEM((_CAT_CH, D), jnp.float32) for _ in range(_NBUF)]
        + [pltpu.VMEM((_TXT_CH, D), jnp.float32) for _ in range(_NBUF)]
        + [pltpu.SemaphoreType.DMA for _ in range(2 * _NBUF)],
    )
    def sc_kernel(cat_idx, txt_idx, wcat, wnlp, out_cat, out_txt,
                  ic_all, it_all, rc0, rc1, rc2, rt0, rt1, rt2,
                  g0, g1, g2, s0, s1, s2):
        wid = lax.axis_index("s") * _NC + lax.axis_index("c")
        cat_base = wid * cat_per_w
        txt_base = wid * txt_per_w
        pltpu.sync_copy(cat_idx.at[pl.ds(cat_base, cat_per_w)], ic_all)
        pltpu.sync_copy(txt_idx.at[pl.ds(txt_base, txt_per_w)], it_all)

        def run_stream(table, idx_v, out, base, ch, n_iters, rows, gsems,
                       ssems):
            def cp_gather(i, s):
                return pltpu.make_async_copy(
                    table.at[idx_v.at[pl.ds(i * ch, ch)]], rows[s], gsems[s])

            def cp_store(i, s):
                return pltpu.make_async_copy(
                    rows[s], out.at[pl.ds(base + i * ch, ch)], ssems[s])

            for k in range(_NBUF):
                cp_gather(k, k).start()

            n_main = max((n_iters - _NBUF) // _NBUF, 0) * _NBUF

            def body(g, carry):
                for j in range(_NBUF):
                    i = g * _NBUF + j
                    cp_gather(i, j).wait()
                    st = cp_store(i, j)
                    st.start()
                    st.wait()
                    cp_gather(i + _NBUF, j).start()
                return carry

            lax.fori_loop(0, n_main // _NBUF, body, 0)
            for i in range(n_main, n_iters):
                s = i % _NBUF
                cp_gather(i, s).wait()
                st = cp_store(i, s)
                st.start()
                st.wait()
                if i + _NBUF < n_iters:
                    cp_gather(i + _NBUF, s).start()

        run_stream(wcat, ic_all, out_cat, cat_base, _CAT_CH, cat_iters,
                   (rc0, rc1, rc2), (g0, g1, g2), (s0, s1, s2))
        run_stream(wnlp, it_all, out_txt, txt_base, _TXT_CH, txt_iters,
                   (rt0, rt1, rt2), (g0, g1, g2), (s0, s1, s2))

    return sc_kernel


def _dense_body(t_ref, w_ref, b_ref, g_ref, og_ref, ot_ref):
    og_ref[...] = jnp.broadcast_to(g_ref[...], og_ref.shape)
    ot_ref[...] = t_ref[...] * w_ref[...] + b_ref[...]


def kernel(target, cat_feat, text, global_token, W_lin, b_lin, W_cat, W_nlp):
    B, S, _ = target.shape
    D = global_token.shape[-1]
    N = B * S
    NT = B * (S + 1)

    gt2 = global_token.reshape(1, D).astype(jnp.float32)
    w2 = W_lin.reshape(1, D).astype(jnp.float32)
    b2 = b_lin.reshape(1, D).astype(jnp.float32)

    cat_idx = cat_feat.reshape(N).astype(jnp.int32)
    wnlp_ext = jnp.concatenate([W_nlp.astype(jnp.float32), gt2], axis=0)
    gt_row = jnp.full((B, 1), W_nlp.shape[0], dtype=jnp.int32)
    txt_idx = jnp.concatenate(
        [gt_row, text.astype(jnp.int32)], axis=1).reshape(NT)

    out_cat_flat, out_txt_flat = _sc_gather_build(N, NT, D)(
        cat_idx, txt_idx, W_cat, wnlp_ext)

    BLK = 2048
    og_flat, ot_flat = pl.pallas_call(
        _dense_body,
        grid=(N // BLK,),
        in_specs=[
            pl.BlockSpec((BLK, 1), lambda i: (i, 0)),
            pl.BlockSpec((1, D), lambda i: (0, 0)),
            pl.BlockSpec((1, D), lambda i: (0, 0)),
            pl.BlockSpec((1, D), lambda i: (0, 0)),
        ],
        out_specs=[
            pl.BlockSpec((BLK, D), lambda i: (i, 0)),
            pl.BlockSpec((BLK, D), lambda i: (i, 0)),
        ],
        out_shape=[
            jax.ShapeDtypeStruct((N, D), jnp.float32),
            jax.ShapeDtypeStruct((N, D), jnp.float32),
        ],
    )(target.reshape(N, 1), w2, b2, gt2)

    out_global = og_flat.reshape(B, S, D)
    out_target = ot_flat.reshape(B, S, D)
    out_cat = out_cat_flat.reshape(B, S, D)
    out_text = out_txt_flat.reshape(B, S + 1, D)
    return (out_global, out_target, out_cat, out_text)
